# baseline (speedup 1.0000x reference)
import functools

import jax
import jax.numpy as jnp
from jax import lax
from jax.experimental import pallas as pl
from jax.experimental.pallas import tpu as pltpu
from jax.experimental.pallas import tpu_sc as plsc

N_NODES = 10000
N_EDGES = 160000
NC = 2
NS = 16

CH = 128
NW = NC * NS
CHUNKS_W = -(-N_EDGES // (CH * NW))
N_EDGES_PAD = CHUNKS_W * CH * NW
DUMP = 64
ROWS_A = 640
ROWS_LAST = N_NODES - 15 * ROWS_A

_MESH = plsc.VectorSubcoreMesh(core_axis_name="c", subcore_axis_name="s")
_SC_PARAMS = pltpu.CompilerParams(use_tc_tiling_on_sc=False)


def _sliced_copy(sid, src_at, dst_at):
    @pl.when(sid < 15)
    def _():
        pltpu.sync_copy(src_at(sid * ROWS_A, ROWS_A), dst_at(sid * ROWS_A, ROWS_A))

    @pl.when(sid == 15)
    def _():
        pltpu.sync_copy(src_at(15 * ROWS_A, ROWS_LAST),
                        dst_at(15 * ROWS_A, ROWS_LAST))


def _fill2d(buf, rows, cols, value):
    @pl.loop(0, rows)
    def _(i):
        @pl.loop(0, cols, step=16)
        def _(c):
            buf[i, pl.ds(c, 16)] = jnp.full((16,), value, jnp.float32)


def _zero_slice(sid, zsrc, dst_at):
    nz = jnp.where(sid < 15, 8, 5)

    @pl.loop(0, nz)
    def _(i):
        pltpu.sync_copy(zsrc, dst_at(sid * ROWS_A + i * 80, 80))


def _sc_agg(dd, S, with_cnt=False, tiled=False):
    NB = 4 if dd <= 64 else 2
    scratch = [
        pltpu.VMEM((CHUNKS_W, CH), jnp.int32),
        pltpu.VMEM((CHUNKS_W, CH), jnp.int32),
    ] + [pltpu.VMEM((CH, dd), jnp.float32) for _ in range(NB)] + [
        pltpu.VMEM_SHARED((N_NODES + DUMP, dd), jnp.float32),
    ] + [pltpu.SemaphoreType.DMA for _ in range(NB)]

    @functools.partial(
        pl.kernel,
        out_type=jax.ShapeDtypeStruct((S, NC, N_NODES, dd), jnp.float32),
        mesh=_MESH, scratch_types=scratch,
        compiler_params=None if tiled else _SC_PARAMS)
    def k(h_hbm, src_hbm, dst_hbm, zeros_hbm, *rest):
        rest = list(rest)
        if with_cnt:
            rest.pop(0)
        out_hbm, src_all, dst_all = rest[:3]
        rows = rest[3:3 + NB]
        acc_sh = rest[3 + NB]
        sems = rest[4 + NB:4 + 2 * NB]
        cid = lax.axis_index("c")
        sid = lax.axis_index("s")
        w = sid * NC + cid
        pltpu.sync_copy(src_hbm.at[pl.ds(w * CHUNKS_W, CHUNKS_W)], src_all)
        pltpu.sync_copy(dst_hbm.at[pl.ds(w * CHUNKS_W, CHUNKS_W)], dst_all)
        for part in range(S):
            _sliced_copy(sid, lambda r, n: zeros_hbm.at[pl.ds(r, n)],
                         lambda r, n: acc_sh.at[pl.ds(r, n)])
            plsc.subcore_barrier()

            def _gather(j, b):
                pltpu.async_copy(h_hbm.at[part].at[src_all.at[j]], rows[b],
                                 sems[b])

            def _gwait(j, b):
                pltpu.make_async_copy(h_hbm.at[part].at[src_all.at[j]],
                                      rows[b], sems[b]).wait()

            for b in range(NB - 1):
                _gather(b, b)

            @pl.loop(0, CHUNKS_W, step=NB)
            def _(i):
                for b in range(NB):
                    j = i + b

                    @pl.when(j + NB - 1 < CHUNKS_W)
                    def _():
                        _gather(j + NB - 1, (b + NB - 1) % NB)

                    _gwait(j, b)
                    pltpu.sync_copy(rows[b], acc_sh.at[dst_all.at[j]],
                                    add=True)

            plsc.subcore_barrier()
            _sliced_copy(sid, lambda r, n: acc_sh.at[pl.ds(r, n)],
                         lambda r, n: out_hbm.at[part].at[cid].at[pl.ds(r, n)])
            if part + 1 < S:
                plsc.subcore_barrier()

    return k


def _sc_cnt():
    @functools.partial(
        pl.kernel,
        out_type=jax.ShapeDtypeStruct((NC, N_NODES, 16), jnp.float32),
        mesh=_MESH,
        scratch_types=[
            pltpu.VMEM((CHUNKS_W, CH), jnp.int32),
            pltpu.VMEM((CH, 16), jnp.float32),
            pltpu.VMEM((80, 16), jnp.float32),
            pltpu.VMEM_SHARED((N_NODES + DUMP, 16), jnp.float32),
        ],
        compiler_params=_SC_PARAMS,
    )
    def k(dst_hbm, out_hbm, dst_all, ones_v, zbuf16, cnt_sh):
        cid = lax.axis_index("c")
        sid = lax.axis_index("s")
        w = sid * NC + cid
        pltpu.sync_copy(dst_hbm.at[pl.ds(w * CHUNKS_W, CHUNKS_W)], dst_all)
        _fill2d(ones_v, CH, 16, 1.0)
        _fill2d(zbuf16, 80, 16, 0.0)
        _zero_slice(sid, zbuf16, lambda r, n: cnt_sh.at[pl.ds(r, n)])
        plsc.subcore_barrier()

        @pl.loop(0, CHUNKS_W)
        def _(i):
            pltpu.sync_copy(ones_v, cnt_sh.at[dst_all.at[i]], add=True)

        plsc.subcore_barrier()
        _sliced_copy(sid, lambda r, n: cnt_sh.at[pl.ds(r, n)],
                     lambda r, n: out_hbm.at[cid].at[pl.ds(r, n)])

    return k


_ROWS_BLK = 2000
_GRID = N_NODES // _ROWS_BLK
def _dot3(x, w):
    xh = x.astype(jnp.bfloat16)
    xl = (x - xh.astype(jnp.float32)).astype(jnp.bfloat16)
    wh = w.astype(jnp.bfloat16)
    wl = (w - wh.astype(jnp.float32)).astype(jnp.bfloat16)
    d = functools.partial(
        jax.lax.dot_general, dimension_numbers=(((1,), (0,)), ((), ())),
        preferred_element_type=jnp.float32)
    return d(xh, wh) + (d(xh, wl) + d(xl, wh))


def _split_store(hl_ref, hl, dd, S, d_real):
    for s in range(S):
        lo = s * dd
        hi = min(d_real, lo + dd)
        part = hl[:, lo:hi]
        if hi - lo < dd:
            part = jnp.concatenate(
                [part,
                 jnp.zeros((part.shape[0], dd - (hi - lo)), jnp.float32)],
                axis=1)
        hl_ref[s] = part


def _gelu(x):
    return x * 0.5 * (1.0 + lax.erf(x * 0.7071067811865476))


def _tc_first(d_in, d_out, S, dd):
    def body(x_ref, wl_ref, wr_ref, b_ref, hl_ref, hr_ref):
        x = x_ref[...]
        hl = _dot3(x, wl_ref[...])
        _split_store(hl_ref, hl, dd, S, d_out)
        hr_ref[...] = _dot3(x, wr_ref[...]) + b_ref[...]

    return pl.pallas_call(
        body,
        grid=(_GRID,),
        in_specs=[
            pl.BlockSpec((_ROWS_BLK, d_in), lambda i: (i, 0)),
            pl.BlockSpec((d_in, d_out), lambda i: (0, 0)),
            pl.BlockSpec((d_in, d_out), lambda i: (0, 0)),
            pl.BlockSpec((1, d_out), lambda i: (0, 0)),
        ],
        out_specs=[
            pl.BlockSpec((S, _ROWS_BLK, dd), lambda i: (0, i, 0)),
            pl.BlockSpec((_ROWS_BLK, d_out), lambda i: (i, 0)),
        ],
        out_shape=[
            jax.ShapeDtypeStruct((S, N_NODES, dd), jnp.float32),
            jax.ShapeDtypeStruct((N_NODES, d_out), jnp.float32),
        ],
    )


def _mean_x(agg_ref, cnt_ref, hrp_ref, d_in, S_in, dd_in):
    c = cnt_ref[0][:, :1] + cnt_ref[1][:, :1]
    inv = 1.0 / jnp.maximum(c, 1.0)
    parts = []
    for s in range(S_in):
        width = min(d_in - s * dd_in, dd_in)
        p = agg_ref[s, 0] + agg_ref[s, 1]
        parts.append(p if width == dd_in else p[:, :width])
    agg = parts[0] if S_in == 1 else jnp.concatenate(parts, axis=1)
    return agg * inv + hrp_ref[...]


def _tc_mid(d_in, d_out, S_in, dd_in, S_out, dd_out):
    def body(agg_ref, cnt_ref, hrp_ref, wl_ref, wr_ref, b_ref, hl_ref, hr_ref):
        x = _gelu(_mean_x(agg_ref, cnt_ref, hrp_ref, d_in, S_in, dd_in))
        hl = _dot3(x, wl_ref[...])
        _split_store(hl_ref, hl, dd_out, S_out, d_out)
        hr_ref[...] = _dot3(x, wr_ref[...]) + b_ref[...]

    return pl.pallas_call(
        body,
        grid=(_GRID,),
        in_specs=[
            pl.BlockSpec((S_in, NC, _ROWS_BLK, dd_in), lambda i: (0, 0, i, 0)),
            pl.BlockSpec((NC, _ROWS_BLK, 16), lambda i: (0, i, 0)),
            pl.BlockSpec((_ROWS_BLK, d_in), lambda i: (i, 0)),
            pl.BlockSpec((d_in, d_out), lambda i: (0, 0)),
            pl.BlockSpec((d_in, d_out), lambda i: (0, 0)),
            pl.BlockSpec((1, d_out), lambda i: (0, 0)),
        ],
        out_specs=[
            pl.BlockSpec((S_out, _ROWS_BLK, dd_out), lambda i: (0, i, 0)),
            pl.BlockSpec((_ROWS_BLK, d_out), lambda i: (i, 0)),
        ],
        out_shape=[
            jax.ShapeDtypeStruct((S_out, N_NODES, dd_out), jnp.float32),
            jax.ShapeDtypeStruct((N_NODES, d_out), jnp.float32),
        ],
    )


def _tc_last(d_in):
    def body(agg_ref, cnt_ref, hrp_ref, out_ref):
        out_ref[...] = _mean_x(agg_ref, cnt_ref, hrp_ref, d_in, 1, d_in)[:, :3]

    return pl.pallas_call(
        body,
        grid=(_GRID,),
        in_specs=[
            pl.BlockSpec((1, NC, _ROWS_BLK, d_in), lambda i: (0, 0, i, 0)),
            pl.BlockSpec((NC, _ROWS_BLK, 16), lambda i: (0, i, 0)),
            pl.BlockSpec((_ROWS_BLK, d_in), lambda i: (i, 0)),
        ],
        out_specs=pl.BlockSpec((_ROWS_BLK, 3), lambda i: (i, 0)),
        out_shape=jax.ShapeDtypeStruct((N_NODES, 3), jnp.float32),
    )


def kernel(features, edges, W_l0, b_l0, W_r0, W_l1, b_l1, W_r1,
           W_l2, b_l2, W_r2, W_l3, b_l3, W_r3):
    e = edges.astype(jnp.int32)
    pad = N_EDGES_PAD - N_EDGES
    fill = jnp.arange(pad, dtype=jnp.int32)
    src = jnp.concatenate([e[0], fill % N_NODES])
    src = src.reshape(NW * CHUNKS_W, CH)
    dst = jnp.concatenate([e[1], N_NODES + (fill % DUMP)])
    dst = dst.reshape(NW * CHUNKS_W, CH)

    wl3 = jnp.pad(W_l3.T.astype(jnp.float32), ((0, 0), (0, 13)))
    wr3 = jnp.pad(W_r3.T.astype(jnp.float32), ((0, 0), (0, 13)))
    b3 = jnp.pad(b_l3.astype(jnp.float32), (0, 13)).reshape(1, 16)

    wls = [W_l0.T, W_l1.T, W_l2.T, wl3]
    wrs = [W_r0.T, W_r1.T, W_r2.T, wr3]
    bs = [b_l0.reshape(1, -1), b_l1.reshape(1, -1), b_l2.reshape(1, -1), b3]
    douts = [192, 128, 64, 16]
    agg_dd = [96, 128, 64, 16]
    agg_S = [2, 1, 1, 1]
    agg_tiled = [False, True, False, False]

    cnt = _sc_cnt()(dst)
    h_l, h_r = _tc_first(256, 192, agg_S[0], agg_dd[0])(
        features, wls[0], wrs[0], bs[0])
    for li in range(1, 4):
        d_prev, d_out = douts[li - 1], douts[li]
        dd, S, tiled = agg_dd[li - 1], agg_S[li - 1], agg_tiled[li - 1]
        zeros = jnp.zeros((N_NODES, dd), jnp.float32)
        if li == 1:
            agg = _sc_agg(dd, S, with_cnt=True, tiled=tiled)(
                h_l, src, dst, zeros, cnt)
        else:
            agg = _sc_agg(dd, S, tiled=tiled)(h_l, src, dst, zeros)
        h_l, h_r = _tc_mid(d_prev, d_out, S, dd, agg_S[li], agg_dd[li])(
            agg, cnt, h_r, wls[li], wrs[li], bs[li])
    agg = _sc_agg(16, 1, tiled=agg_tiled[3])(
        h_l, src, dst, jnp.zeros((N_NODES, 16), jnp.float32))
    return _tc_last(16)(agg, cnt, h_r)


# speedup vs baseline: 1.0039x; 1.0039x over previous
import functools

import jax
import jax.numpy as jnp
from jax import lax
from jax.experimental import pallas as pl
from jax.experimental.pallas import tpu as pltpu
from jax.experimental.pallas import tpu_sc as plsc

N_NODES = 10000
N_EDGES = 160000
NC = 2
NS = 16

CH = 128
NW = NC * NS
CHUNKS_W = -(-N_EDGES // (CH * NW))
N_EDGES_PAD = CHUNKS_W * CH * NW
DUMP = 64
ROWS_A = 640
ROWS_LAST = N_NODES - 15 * ROWS_A

_MESH = plsc.VectorSubcoreMesh(core_axis_name="c", subcore_axis_name="s")
_SC_PARAMS = pltpu.CompilerParams(use_tc_tiling_on_sc=False)


def _sliced_copy(sid, src_at, dst_at):
    @pl.when(sid < 15)
    def _():
        pltpu.sync_copy(src_at(sid * ROWS_A, ROWS_A), dst_at(sid * ROWS_A, ROWS_A))

    @pl.when(sid == 15)
    def _():
        pltpu.sync_copy(src_at(15 * ROWS_A, ROWS_LAST),
                        dst_at(15 * ROWS_A, ROWS_LAST))


def _fill2d(buf, rows, cols, value):
    @pl.loop(0, rows)
    def _(i):
        @pl.loop(0, cols, step=16)
        def _(c):
            buf[i, pl.ds(c, 16)] = jnp.full((16,), value, jnp.float32)


def _zero_slice(sid, zsrc, dst_at):
    nz = jnp.where(sid < 15, 8, 5)

    @pl.loop(0, nz)
    def _(i):
        pltpu.sync_copy(zsrc, dst_at(sid * ROWS_A + i * 80, 80))


def _sc_agg(dd, S, with_cnt=False, tiled=False):
    NB = 4 if dd <= 64 else 2
    scratch = [
        pltpu.VMEM((CHUNKS_W, CH), jnp.int32),
        pltpu.VMEM((CHUNKS_W, CH), jnp.int32),
    ] + [pltpu.VMEM((CH, dd), jnp.float32) for _ in range(NB)] + [
        pltpu.VMEM_SHARED((N_NODES + DUMP, dd), jnp.float32),
    ] + [pltpu.SemaphoreType.DMA for _ in range(NB)]

    @functools.partial(
        pl.kernel,
        out_type=jax.ShapeDtypeStruct((S, NC, N_NODES, dd), jnp.float32),
        mesh=_MESH, scratch_types=scratch,
        compiler_params=None if tiled else _SC_PARAMS)
    def k(h_hbm, src_hbm, dst_hbm, zeros_hbm, *rest):
        rest = list(rest)
        if with_cnt:
            rest.pop(0)
        out_hbm, src_all, dst_all = rest[:3]
        rows = rest[3:3 + NB]
        acc_sh = rest[3 + NB]
        sems = rest[4 + NB:4 + 2 * NB]
        cid = lax.axis_index("c")
        sid = lax.axis_index("s")
        w = sid * NC + cid
        pltpu.sync_copy(src_hbm.at[pl.ds(w * CHUNKS_W, CHUNKS_W)], src_all)
        pltpu.sync_copy(dst_hbm.at[pl.ds(w * CHUNKS_W, CHUNKS_W)], dst_all)
        for part in range(S):
            _sliced_copy(sid, lambda r, n: zeros_hbm.at[pl.ds(r, n)],
                         lambda r, n: acc_sh.at[pl.ds(r, n)])
            plsc.subcore_barrier()

            def _gather(j, b):
                pltpu.async_copy(h_hbm.at[part].at[src_all.at[j]], rows[b],
                                 sems[b])

            def _gwait(j, b):
                pltpu.make_async_copy(h_hbm.at[part].at[src_all.at[j]],
                                      rows[b], sems[b]).wait()

            for b in range(NB - 1):
                _gather(b, b)

            @pl.loop(0, CHUNKS_W, step=NB)
            def _(i):
                for b in range(NB):
                    j = i + b

                    @pl.when(j + NB - 1 < CHUNKS_W)
                    def _():
                        _gather(j + NB - 1, (b + NB - 1) % NB)

                    _gwait(j, b)
                    pltpu.sync_copy(rows[b], acc_sh.at[dst_all.at[j]],
                                    add=True)

            plsc.subcore_barrier()
            _sliced_copy(sid, lambda r, n: acc_sh.at[pl.ds(r, n)],
                         lambda r, n: out_hbm.at[part].at[cid].at[pl.ds(r, n)])
            if part + 1 < S:
                plsc.subcore_barrier()

    return k


def _sc_cnt():
    @functools.partial(
        pl.kernel,
        out_type=jax.ShapeDtypeStruct((NC, N_NODES, 16), jnp.float32),
        mesh=_MESH,
        scratch_types=[
            pltpu.VMEM((CHUNKS_W, CH), jnp.int32),
            pltpu.VMEM((CH, 16), jnp.float32),
            pltpu.VMEM((80, 16), jnp.float32),
            pltpu.VMEM_SHARED((N_NODES + DUMP, 16), jnp.float32),
        ],
        compiler_params=_SC_PARAMS,
    )
    def k(dst_hbm, out_hbm, dst_all, ones_v, zbuf16, cnt_sh):
        cid = lax.axis_index("c")
        sid = lax.axis_index("s")
        w = sid * NC + cid
        pltpu.sync_copy(dst_hbm.at[pl.ds(w * CHUNKS_W, CHUNKS_W)], dst_all)
        _fill2d(ones_v, CH, 16, 1.0)
        _fill2d(zbuf16, 80, 16, 0.0)
        _zero_slice(sid, zbuf16, lambda r, n: cnt_sh.at[pl.ds(r, n)])
        plsc.subcore_barrier()

        @pl.loop(0, CHUNKS_W)
        def _(i):
            pltpu.sync_copy(ones_v, cnt_sh.at[dst_all.at[i]], add=True)

        plsc.subcore_barrier()
        _sliced_copy(sid, lambda r, n: cnt_sh.at[pl.ds(r, n)],
                     lambda r, n: out_hbm.at[cid].at[pl.ds(r, n)])

    return k


_ROWS_BLK = 2000
_GRID = N_NODES // _ROWS_BLK
def _dot3(x, w):
    xh = x.astype(jnp.bfloat16)
    xl = (x - xh.astype(jnp.float32)).astype(jnp.bfloat16)
    wh = w.astype(jnp.bfloat16)
    wl = (w - wh.astype(jnp.float32)).astype(jnp.bfloat16)
    d = functools.partial(
        jax.lax.dot_general, dimension_numbers=(((1,), (0,)), ((), ())),
        preferred_element_type=jnp.float32)
    return d(xh, wh) + (d(xh, wl) + d(xl, wh))


def _split_store(hl_ref, hl, dd, S, d_real):
    for s in range(S):
        lo = s * dd
        hi = min(d_real, lo + dd)
        part = hl[:, lo:hi]
        if hi - lo < dd:
            part = jnp.concatenate(
                [part,
                 jnp.zeros((part.shape[0], dd - (hi - lo)), jnp.float32)],
                axis=1)
        hl_ref[s] = part


def _gelu(x):
    return x * 0.5 * (1.0 + lax.erf(x * 0.7071067811865476))


def _tc_first(d_in, d_out, S, dd):
    def body(x_ref, wl_ref, wr_ref, b_ref, hl_ref, hr_ref):
        x = x_ref[...]
        hl = _dot3(x, wl_ref[...])
        _split_store(hl_ref, hl, dd, S, d_out)
        hr_ref[...] = _dot3(x, wr_ref[...]) + b_ref[...]

    return pl.pallas_call(
        body,
        grid=(_GRID,),
        in_specs=[
            pl.BlockSpec((_ROWS_BLK, d_in), lambda i: (i, 0)),
            pl.BlockSpec((d_in, d_out), lambda i: (0, 0)),
            pl.BlockSpec((d_in, d_out), lambda i: (0, 0)),
            pl.BlockSpec((1, d_out), lambda i: (0, 0)),
        ],
        out_specs=[
            pl.BlockSpec((S, _ROWS_BLK, dd), lambda i: (0, i, 0)),
            pl.BlockSpec((_ROWS_BLK, d_out), lambda i: (i, 0)),
        ],
        out_shape=[
            jax.ShapeDtypeStruct((S, N_NODES, dd), jnp.float32),
            jax.ShapeDtypeStruct((N_NODES, d_out), jnp.float32),
        ],
    )


def _mean_x(agg_ref, cnt_ref, hrp_ref, d_in, S_in, dd_in):
    c = cnt_ref[0][:, :1] + cnt_ref[1][:, :1]
    inv = 1.0 / jnp.maximum(c, 1.0)
    parts = []
    for s in range(S_in):
        width = min(d_in - s * dd_in, dd_in)
        p = agg_ref[s, 0] + agg_ref[s, 1]
        parts.append(p if width == dd_in else p[:, :width])
    agg = parts[0] if S_in == 1 else jnp.concatenate(parts, axis=1)
    return agg * inv + hrp_ref[...]


def _tc_mid(d_in, d_out, S_in, dd_in, S_out, dd_out):
    def body(agg_ref, cnt_ref, hrp_ref, wl_ref, wr_ref, b_ref, hl_ref, hr_ref):
        x = _gelu(_mean_x(agg_ref, cnt_ref, hrp_ref, d_in, S_in, dd_in))
        hl = _dot3(x, wl_ref[...])
        _split_store(hl_ref, hl, dd_out, S_out, d_out)
        hr_ref[...] = _dot3(x, wr_ref[...]) + b_ref[...]

    return pl.pallas_call(
        body,
        grid=(_GRID,),
        in_specs=[
            pl.BlockSpec((S_in, NC, _ROWS_BLK, dd_in), lambda i: (0, 0, i, 0)),
            pl.BlockSpec((NC, _ROWS_BLK, 16), lambda i: (0, i, 0)),
            pl.BlockSpec((_ROWS_BLK, d_in), lambda i: (i, 0)),
            pl.BlockSpec((d_in, d_out), lambda i: (0, 0)),
            pl.BlockSpec((d_in, d_out), lambda i: (0, 0)),
            pl.BlockSpec((1, d_out), lambda i: (0, 0)),
        ],
        out_specs=[
            pl.BlockSpec((S_out, _ROWS_BLK, dd_out), lambda i: (0, i, 0)),
            pl.BlockSpec((_ROWS_BLK, d_out), lambda i: (i, 0)),
        ],
        out_shape=[
            jax.ShapeDtypeStruct((S_out, N_NODES, dd_out), jnp.float32),
            jax.ShapeDtypeStruct((N_NODES, d_out), jnp.float32),
        ],
    )


def _tc_last(d_in):
    def body(agg_ref, cnt_ref, hrp_ref, out_ref):
        out_ref[...] = _mean_x(agg_ref, cnt_ref, hrp_ref, d_in, 1, d_in)[:, :3]

    return pl.pallas_call(
        body,
        grid=(_GRID,),
        in_specs=[
            pl.BlockSpec((1, NC, _ROWS_BLK, d_in), lambda i: (0, 0, i, 0)),
            pl.BlockSpec((NC, _ROWS_BLK, 16), lambda i: (0, i, 0)),
            pl.BlockSpec((_ROWS_BLK, d_in), lambda i: (i, 0)),
        ],
        out_specs=pl.BlockSpec((_ROWS_BLK, 3), lambda i: (i, 0)),
        out_shape=jax.ShapeDtypeStruct((N_NODES, 3), jnp.float32),
    )


def kernel(features, edges, W_l0, b_l0, W_r0, W_l1, b_l1, W_r1,
           W_l2, b_l2, W_r2, W_l3, b_l3, W_r3):
    e = edges.astype(jnp.int32)
    pad = N_EDGES_PAD - N_EDGES
    fill = jnp.arange(pad, dtype=jnp.int32)
    src = jnp.concatenate([e[0], fill % N_NODES])
    src = src.reshape(NW * CHUNKS_W, CH)
    dst = jnp.concatenate([e[1], N_NODES + (fill % DUMP)])
    dst = dst.reshape(NW * CHUNKS_W, CH)

    wl3 = jnp.pad(W_l3.T.astype(jnp.float32), ((0, 0), (0, 13)))
    wr3 = jnp.pad(W_r3.T.astype(jnp.float32), ((0, 0), (0, 13)))
    b3 = jnp.pad(b_l3.astype(jnp.float32), (0, 13)).reshape(1, 16)

    wls = [W_l0.T, W_l1.T, W_l2.T, wl3]
    wrs = [W_r0.T, W_r1.T, W_r2.T, wr3]
    bs = [b_l0.reshape(1, -1), b_l1.reshape(1, -1), b_l2.reshape(1, -1), b3]
    douts = [192, 128, 64, 16]
    agg_dd = [128, 128, 64, 16]
    agg_S = [2, 1, 1, 1]
    agg_tiled = [True, True, False, False]

    cnt = _sc_cnt()(dst)
    h_l, h_r = _tc_first(256, 192, agg_S[0], agg_dd[0])(
        features, wls[0], wrs[0], bs[0])
    for li in range(1, 4):
        d_prev, d_out = douts[li - 1], douts[li]
        dd, S, tiled = agg_dd[li - 1], agg_S[li - 1], agg_tiled[li - 1]
        zeros = jnp.zeros((N_NODES, dd), jnp.float32)
        if li == 1:
            agg = _sc_agg(dd, S, with_cnt=True, tiled=tiled)(
                h_l, src, dst, zeros, cnt)
        else:
            agg = _sc_agg(dd, S, tiled=tiled)(h_l, src, dst, zeros)
        h_l, h_r = _tc_mid(d_prev, d_out, S, dd, agg_S[li], agg_dd[li])(
            agg, cnt, h_r, wls[li], wrs[li], bs[li])
    agg = _sc_agg(16, 1, tiled=agg_tiled[3])(
        h_l, src, dst, jnp.zeros((N_NODES, 16), jnp.float32))
    return _tc_last(16)(agg, cnt, h_r)
